# baseline (speedup 1.0000x reference)
import functools

import jax
import jax.numpy as jnp
from jax import lax
from jax.experimental import pallas as pl
from jax.experimental.pallas import tpu as pltpu
from jax.experimental.pallas import tpu_sc as plsc

L = 16

OFF_CARD = 1001
OFF_POTION = 1033
OFF_PATH = 1073
OFF_LINKS = 1163
OFF_BOSS = 1283
OFF_SCAL = 1299
OUT_LEN = 1302
OUT_PAD = 1312

N_DECK = 100
DECK_PAD = 112
COLL_PAD = 1008
TAIL_PAD = 128
BOSS_ID_POS = 123


def _body(coll_h, deck_h, potion_h, path_h, tail_h, pot_emb_h, node_emb_h,
          boss_emb_h, card_emb_h, out_h,
          coll_v, deck_v, potion_v, path_v, tail_v, pot_emb_v, node_emb_v,
          boss_emb_v, cards_v, out_v, sem, gsem):
    @pl.when(jnp.logical_and(lax.axis_index("c") == 0, lax.axis_index("s") == 0))
    def _():
        copies = [
            pltpu.async_copy(deck_h, deck_v, sem),
            pltpu.async_copy(coll_h, coll_v, sem),
            pltpu.async_copy(potion_h, potion_v, sem),
            pltpu.async_copy(path_h, path_v, sem),
            pltpu.async_copy(tail_h, tail_v, sem),
            pltpu.async_copy(pot_emb_h, pot_emb_v, sem),
            pltpu.async_copy(node_emb_h, node_emb_v, sem),
            pltpu.async_copy(boss_emb_h, boss_emb_v, sem),
        ]
        for c in copies:
            c.wait()

        gather = pltpu.async_copy(card_emb_h.at[deck_v], cards_v, gsem)

        lanes = lax.iota(jnp.int32, L)

        for i in range(62):
            out_v[pl.ds(i * L, L)] = coll_v[pl.ds(i * L, L)].astype(jnp.float32)
        t = 62 * L + lanes
        plsc.store_scatter(out_v, [t],
                           coll_v[pl.ds(62 * L, L)].astype(jnp.float32),
                           mask=t < OFF_CARD)

        for c in range(3):
            t = c * L + lanes
            row = plsc.load_gather(potion_v, [t // 8])
            val = plsc.load_gather(pot_emb_v, [row * 8 + t % 8])
            plsc.store_scatter(out_v, [OFF_POTION + t], val, mask=t < 40)

        for c in range(6):
            t = c * L + lanes
            row = plsc.load_gather(path_v, [t // 6])
            val = plsc.load_gather(node_emb_v, [row * 6 + t % 6])
            plsc.store_scatter(out_v, [OFF_PATH + t], val, mask=t < 90)

        for c in range(8):
            t = c * L + lanes
            val = tail_v[pl.ds(c * L, L)].astype(jnp.float32)
            idx = jnp.where(t < 120, OFF_LINKS + t, OFF_SCAL + (t - 120))
            plsc.store_scatter(out_v, [idx], val, mask=t < 123)

        boss = plsc.load_gather(tail_v, [jnp.full((L,), BOSS_ID_POS, jnp.int32)])
        bval = plsc.load_gather(boss_emb_v, [boss * L + lanes])
        plsc.store_scatter(out_v, [OFF_BOSS + lanes], bval)

        gather.wait()

        def acc(i, carry):
            a0, a1 = carry
            return (a0 + cards_v[i, pl.ds(0, L)], a1 + cards_v[i, pl.ds(L, L)])

        zero = jnp.zeros((L,), jnp.float32)
        a0, a1 = lax.fori_loop(0, N_DECK, acc, (zero, zero))
        scale = jnp.float32(1.0 / N_DECK)
        plsc.store_scatter(out_v, [OFF_CARD + lanes], a0 * scale)
        plsc.store_scatter(out_v, [OFF_CARD + L + lanes], a1 * scale)

        pltpu.sync_copy(out_v, out_h)


_encode = pl.kernel(
    _body,
    out_type=jax.ShapeDtypeStruct((OUT_PAD,), jnp.float32),
    mesh=plsc.VectorSubcoreMesh(core_axis_name="c", subcore_axis_name="s",
                                num_cores=2, num_subcores=16),
    compiler_params=pltpu.CompilerParams(needs_layout_passes=False,
                                         use_tc_tiling_on_sc=False),
    scratch_types=[
        pltpu.VMEM((COLL_PAD,), jnp.int32),
        pltpu.VMEM((DECK_PAD,), jnp.int32),
        pltpu.VMEM((L,), jnp.int32),
        pltpu.VMEM((L,), jnp.int32),
        pltpu.VMEM((TAIL_PAD,), jnp.int32),
        pltpu.VMEM((416,), jnp.float32),
        pltpu.VMEM((64,), jnp.float32),
        pltpu.VMEM((320,), jnp.float32),
        pltpu.VMEM((DECK_PAD, 32), jnp.float32),
        pltpu.VMEM((OUT_PAD,), jnp.float32),
        pltpu.SemaphoreType.DMA,
        pltpu.SemaphoreType.DMA,
    ],
)


def kernel(collection, card_deck, potion, path_nodes, current_links,
           next_links, boss_id, current_health, max_health, current_floor,
           card_embed, potion_embed, node_embed, boss_embed):
    i32 = jnp.int32
    coll_p = jnp.pad(collection.astype(i32), (0, COLL_PAD - 1001))
    deck_p = jnp.pad(card_deck.astype(i32), (0, DECK_PAD - N_DECK))
    potion_p = jnp.pad(potion.astype(i32), (0, L - 5))
    path_p = jnp.pad(path_nodes.astype(i32), (0, L - 15))
    tail = jnp.concatenate([
        current_links.astype(i32).reshape(-1),
        next_links.astype(i32).reshape(-1),
        jnp.stack([jnp.asarray(current_health, i32),
                   jnp.asarray(max_health, i32),
                   jnp.asarray(current_floor, i32),
                   jnp.asarray(boss_id, i32)]),
    ])
    tail_p = jnp.pad(tail, (0, TAIL_PAD - 124))
    pot_emb_p = jnp.pad(potion_embed.reshape(-1), (0, 416 - 408))
    node_emb_p = jnp.pad(node_embed.reshape(-1), (0, 64 - 60))
    boss_emb_f = boss_embed.reshape(-1)
    out = _encode(coll_p, deck_p, potion_p, path_p, tail_p, pot_emb_p,
                  node_emb_p, boss_emb_f, card_embed)
    return out[:OUT_LEN]


# speedup vs baseline: 1.0531x; 1.0531x over previous
import functools

import jax
import jax.numpy as jnp
from jax import lax
from jax.experimental import pallas as pl
from jax.experimental.pallas import tpu as pltpu
from jax.experimental.pallas import tpu_sc as plsc

L = 16

OFF_CARD = 1001
OFF_POTION = 1033
OFF_PATH = 1073
OFF_LINKS = 1163
OFF_BOSS = 1283
OFF_SCAL = 1299
OUT_LEN = 1302
OUT_PAD = 1312

N_DECK = 100
DECK_PAD = 112
COLL_PAD = 1008
TAIL_PAD = 128
BOSS_ID_POS = 123


def _body(coll_h, deck_h, potion_h, path_h, tail_h, pot_emb_h, node_emb_h,
          boss_emb_h, card_emb_h, out_h,
          coll_v, deck_v, potion_v, path_v, tail_v, pot_emb_v, node_emb_v,
          boss_emb_v, cards_v, out_v, sem, gsem):
    @pl.when(jnp.logical_and(lax.axis_index("c") == 0, lax.axis_index("s") == 0))
    def _():
        copies = [
            pltpu.async_copy(deck_h, deck_v, sem),
            pltpu.async_copy(coll_h, coll_v, sem),
            pltpu.async_copy(potion_h, potion_v, sem),
            pltpu.async_copy(path_h, path_v, sem),
            pltpu.async_copy(tail_h, tail_v, sem),
            pltpu.async_copy(pot_emb_h, pot_emb_v, sem),
            pltpu.async_copy(node_emb_h, node_emb_v, sem),
            pltpu.async_copy(boss_emb_h, boss_emb_v, sem),
        ]
        for c in copies:
            c.wait()

        gather = pltpu.async_copy(card_emb_h.at[deck_v], cards_v, gsem)

        lanes = lax.iota(jnp.int32, L)

        for i in range(62):
            out_v[pl.ds(i * L, L)] = coll_v[pl.ds(i * L, L)].astype(jnp.float32)
        t = 62 * L + lanes
        plsc.store_scatter(out_v, [t],
                           coll_v[pl.ds(62 * L, L)].astype(jnp.float32),
                           mask=t < OFF_CARD)

        for c in range(3):
            t = c * L + lanes
            row = plsc.load_gather(potion_v, [t // 8])
            val = plsc.load_gather(pot_emb_v, [row * 8 + t % 8])
            plsc.store_scatter(out_v, [OFF_POTION + t], val, mask=t < 40)

        for c in range(6):
            t = c * L + lanes
            row = plsc.load_gather(path_v, [t // 6])
            val = plsc.load_gather(node_emb_v, [row * 6 + t % 6])
            plsc.store_scatter(out_v, [OFF_PATH + t], val, mask=t < 90)

        for c in range(8):
            t = c * L + lanes
            val = tail_v[pl.ds(c * L, L)].astype(jnp.float32)
            idx = jnp.where(t < 120, OFF_LINKS + t, OFF_SCAL + (t - 120))
            plsc.store_scatter(out_v, [idx], val, mask=t < 123)

        boss = plsc.load_gather(tail_v, [jnp.full((L,), BOSS_ID_POS, jnp.int32)])
        bval = plsc.load_gather(boss_emb_v, [boss * L + lanes])
        plsc.store_scatter(out_v, [OFF_BOSS + lanes], bval)

        gather.wait()

        def acc(i, carry):
            a0, a1 = carry
            return (a0 + cards_v[i, pl.ds(0, L)], a1 + cards_v[i, pl.ds(L, L)])

        zero = jnp.zeros((L,), jnp.float32)
        a0, a1 = lax.fori_loop(0, N_DECK, acc, (zero, zero))
        scale = jnp.float32(1.0 / N_DECK)
        plsc.store_scatter(out_v, [OFF_CARD + lanes], a0 * scale)
        plsc.store_scatter(out_v, [OFF_CARD + L + lanes], a1 * scale)

        pltpu.sync_copy(out_v, out_h)


_encode = pl.kernel(
    _body,
    out_type=jax.ShapeDtypeStruct((OUT_PAD,), jnp.float32),
    mesh=plsc.VectorSubcoreMesh(core_axis_name="c", subcore_axis_name="s",
                                num_cores=1, num_subcores=16),
    compiler_params=pltpu.CompilerParams(needs_layout_passes=False,
                                         use_tc_tiling_on_sc=False,
                                         skip_device_barrier=True),
    scratch_types=[
        pltpu.VMEM((COLL_PAD,), jnp.int32),
        pltpu.VMEM((DECK_PAD,), jnp.int32),
        pltpu.VMEM((L,), jnp.int32),
        pltpu.VMEM((L,), jnp.int32),
        pltpu.VMEM((TAIL_PAD,), jnp.int32),
        pltpu.VMEM((416,), jnp.float32),
        pltpu.VMEM((64,), jnp.float32),
        pltpu.VMEM((320,), jnp.float32),
        pltpu.VMEM((DECK_PAD, 32), jnp.float32),
        pltpu.VMEM((OUT_PAD,), jnp.float32),
        pltpu.SemaphoreType.DMA,
        pltpu.SemaphoreType.DMA,
    ],
)


def kernel(collection, card_deck, potion, path_nodes, current_links,
           next_links, boss_id, current_health, max_health, current_floor,
           card_embed, potion_embed, node_embed, boss_embed):
    i32 = jnp.int32
    coll_p = jnp.pad(collection.astype(i32), (0, COLL_PAD - 1001))
    deck_p = jnp.pad(card_deck.astype(i32), (0, DECK_PAD - N_DECK))
    potion_p = jnp.pad(potion.astype(i32), (0, L - 5))
    path_p = jnp.pad(path_nodes.astype(i32), (0, L - 15))
    tail = jnp.concatenate([
        current_links.astype(i32).reshape(-1),
        next_links.astype(i32).reshape(-1),
        jnp.stack([jnp.asarray(current_health, i32),
                   jnp.asarray(max_health, i32),
                   jnp.asarray(current_floor, i32),
                   jnp.asarray(boss_id, i32)]),
    ])
    tail_p = jnp.pad(tail, (0, TAIL_PAD - 124))
    pot_emb_p = jnp.pad(potion_embed.reshape(-1), (0, 416 - 408))
    node_emb_p = jnp.pad(node_embed.reshape(-1), (0, 64 - 60))
    boss_emb_f = boss_embed.reshape(-1)
    out = _encode(coll_p, deck_p, potion_p, path_p, tail_p, pot_emb_p,
                  node_emb_p, boss_emb_f, card_embed)
    return out[:OUT_LEN]
